# baseline (speedup 1.0000x reference)
import functools

import jax
import jax.numpy as jnp
from jax import lax
from jax.experimental import pallas as pl
from jax.experimental.pallas import tpu as pltpu
from jax.experimental.pallas import tpu_sc as plsc

BATCH = 16384
VOCAB = 1000000
EMB = 64
LANES = 16
WIN = 128

NC = 2
NS = 16
NW = NC * NS
B_PER_W = BATCH // NW
SG = 16
N_SG = B_PER_W // SG
N_PAIRS = SG // 2


def _make_kernel():
    mesh = plsc.VectorSubcoreMesh(core_axis_name="c", subcore_axis_name="s")

    @functools.partial(
        pl.kernel,
        mesh=mesh,
        out_type=jax.ShapeDtypeStruct((EMB, BATCH), jnp.float32),
        scratch_types=[
            pltpu.VMEM((B_PER_W,), jnp.int32),
            pltpu.VMEM((B_PER_W,), jnp.int32),
            pltpu.VMEM((4, EMB, WIN), jnp.float32),
            pltpu.VMEM((4, EMB, WIN), jnp.float32),
            pltpu.VMEM((EMB, B_PER_W), jnp.float32),
            pltpu.SemaphoreType.DMA((4, 2)),
            pltpu.SemaphoreType.DMA((4, 2)),
        ],
        compiler_params=pltpu.CompilerParams(needs_layout_passes=False),
    )
    def k(tbl_a_t, tbl_b_t, idx_a_hbm, idx_b_hbm, out_t_hbm,
          idx_a_v, idx_b_v, blk_a, blk_b, out_v, sem_a, sem_b):
        wid = lax.axis_index("s") * NC + lax.axis_index("c")
        base = wid * B_PER_W

        pltpu.sync_copy(idx_a_hbm.at[pl.ds(base, B_PER_W)], idx_a_v)
        pltpu.sync_copy(idx_b_hbm.at[pl.ds(base, B_PER_W)], idx_b_v)

        lane_ids = lax.iota(jnp.int32, LANES)

        DEPTH = 3

        HE = EMB // 2

        def fire_one(slot, ra, rb):
            ca = pl.multiple_of((ra >> 7) * WIN, WIN)
            cb = pl.multiple_of((rb >> 7) * WIN, WIN)
            for half in range(2):
                fs = pl.ds(half * HE, HE)
                pltpu.async_copy(
                    tbl_a_t.at[fs, pl.ds(ca, WIN)], blk_a.at[slot, fs],
                    sem_a.at[slot, half])
                pltpu.async_copy(
                    tbl_b_t.at[fs, pl.ds(cb, WIN)], blk_b.at[slot, fs],
                    sem_b.at[slot, half])

        def drain(slot):
            for half in range(2):
                fs = pl.ds(half * HE, HE)
                pltpu.make_async_copy(
                    tbl_a_t.at[fs, pl.ds(0, WIN)], blk_a.at[slot, fs],
                    sem_a.at[slot, half]).wait()
                pltpu.make_async_copy(
                    tbl_b_t.at[fs, pl.ds(0, WIN)], blk_b.at[slot, fs],
                    sem_b.at[slot, half]).wait()

        def process(slot, ra, rb, row):
            la = jnp.full((LANES,), ra & (WIN - 1), jnp.int32)
            lb = jnp.full((LANES,), rb & (WIN - 1), jnp.int32)
            col = jnp.full((LANES,), row, jnp.int32)
            for c4 in range(EMB // LANES):
                feat = lane_ids + (c4 * LANES)
                ea = plsc.load_gather(blk_a.at[slot], [feat, la])
                eb = plsc.load_gather(blk_b.at[slot], [feat, lb])
                plsc.store_scatter(out_v, [feat, col], ea * eb)

        va0 = idx_a_v[pl.ds(0, SG)]
        vb0 = idx_b_v[pl.ds(0, SG)]
        for j in range(DEPTH):
            fire_one(j & 3, va0[j], vb0[j])

        def body(sg, carry):
            va = idx_a_v[pl.ds(sg * SG, SG)]
            vb = idx_b_v[pl.ds(sg * SG, SG)]
            nxt = jnp.minimum(sg + 1, N_SG - 1) * SG
            va_n = idx_a_v[pl.ds(nxt, SG)]
            vb_n = idx_b_v[pl.ds(nxt, SG)]
            for j in range(SG):
                jn = j + DEPTH
                if jn < SG:
                    fire_one(jn & 3, va[jn], vb[jn])
                else:
                    @pl.when(sg < N_SG - 1)
                    def _():
                        fire_one(jn & 3, va_n[jn - SG], vb_n[jn - SG])
                drain(j & 3)
                process(j & 3, va[j], vb[j], sg * SG + j)
            return carry

        lax.fori_loop(0, N_SG, body, 0)

        pltpu.sync_copy(out_v, out_t_hbm.at[:, pl.ds(base, B_PER_W)])

    return k


_sc_kernel = _make_kernel()


def kernel(inputs, emb_inst_table, emb_cont_table):
    idx_i = inputs[:, 0].astype(jnp.int32)
    idx_c = inputs[:, 1].astype(jnp.int32)
    out_t = _sc_kernel(emb_inst_table.T, emb_cont_table.T, idx_i, idx_c)
    return out_t.T


# speedup vs baseline: 1.0933x; 1.0933x over previous
import functools

import jax
import jax.numpy as jnp
from jax import lax
from jax.experimental import pallas as pl
from jax.experimental.pallas import tpu as pltpu
from jax.experimental.pallas import tpu_sc as plsc

BATCH = 16384
VOCAB = 1000000
EMB = 64
LANES = 16
WIN = 128

NC = 2
NS = 16
NW = NC * NS
B_PER_W = BATCH // NW
SG = 16
N_SG = B_PER_W // SG
N_PAIRS = SG // 2


def _make_kernel():
    mesh = plsc.VectorSubcoreMesh(core_axis_name="c", subcore_axis_name="s")

    @functools.partial(
        pl.kernel,
        mesh=mesh,
        out_type=jax.ShapeDtypeStruct((EMB, BATCH), jnp.float32),
        scratch_types=[
            pltpu.VMEM((B_PER_W,), jnp.int32),
            pltpu.VMEM((B_PER_W,), jnp.int32),
            pltpu.VMEM((4, EMB, WIN), jnp.float32),
            pltpu.VMEM((4, EMB, WIN), jnp.float32),
            pltpu.VMEM((EMB, B_PER_W), jnp.float32),
            pltpu.SemaphoreType.DMA((4,)),
            pltpu.SemaphoreType.DMA((4,)),
        ],
        compiler_params=pltpu.CompilerParams(needs_layout_passes=False),
    )
    def k(tbl_a_t, tbl_b_t, idx_t_hbm, out_t_hbm,
          idx_a_v, idx_b_v, blk_a, blk_b, out_v, sem_a, sem_b):
        wid = lax.axis_index("s") * NC + lax.axis_index("c")
        base = wid * B_PER_W

        pltpu.sync_copy(idx_t_hbm.at[0, pl.ds(base, B_PER_W)], idx_a_v)
        pltpu.sync_copy(idx_t_hbm.at[1, pl.ds(base, B_PER_W)], idx_b_v)

        lane_ids = lax.iota(jnp.int32, LANES)

        DEPTH = 3

        def fire_one(slot, ra, rb):
            ca = pl.multiple_of((ra >> 7) * WIN, WIN)
            cb = pl.multiple_of((rb >> 7) * WIN, WIN)
            pltpu.async_copy(
                tbl_a_t.at[:, pl.ds(ca, WIN)], blk_a.at[slot],
                sem_a.at[slot])
            pltpu.async_copy(
                tbl_b_t.at[:, pl.ds(cb, WIN)], blk_b.at[slot],
                sem_b.at[slot])

        def drain(slot):
            pltpu.make_async_copy(
                tbl_a_t.at[:, pl.ds(0, WIN)], blk_a.at[slot],
                sem_a.at[slot]).wait()
            pltpu.make_async_copy(
                tbl_b_t.at[:, pl.ds(0, WIN)], blk_b.at[slot],
                sem_b.at[slot]).wait()

        def process(slot, ra, rb, row):
            la = jnp.full((LANES,), ra & (WIN - 1), jnp.int32)
            lb = jnp.full((LANES,), rb & (WIN - 1), jnp.int32)
            col = jnp.full((LANES,), row, jnp.int32)
            for c4 in range(EMB // LANES):
                feat = lane_ids + (c4 * LANES)
                ea = plsc.load_gather(blk_a.at[slot], [feat, la])
                eb = plsc.load_gather(blk_b.at[slot], [feat, lb])
                plsc.store_scatter(out_v, [feat, col], ea * eb)

        va0 = idx_a_v[pl.ds(0, SG)]
        vb0 = idx_b_v[pl.ds(0, SG)]
        for j in range(DEPTH):
            fire_one(j & 3, va0[j], vb0[j])

        def body(sg, carry):
            va = idx_a_v[pl.ds(sg * SG, SG)]
            vb = idx_b_v[pl.ds(sg * SG, SG)]
            nxt = jnp.minimum(sg + 1, N_SG - 1) * SG
            va_n = idx_a_v[pl.ds(nxt, SG)]
            vb_n = idx_b_v[pl.ds(nxt, SG)]
            for j in range(SG):
                jn = j + DEPTH
                if jn < SG:
                    fire_one(jn & 3, va[jn], vb[jn])
                else:
                    @pl.when(sg < N_SG - 1)
                    def _():
                        fire_one(jn & 3, va_n[jn - SG], vb_n[jn - SG])
                drain(j & 3)
                process(j & 3, va[j], vb[j], sg * SG + j)
            return carry

        lax.fori_loop(0, N_SG, body, 0)

        pltpu.sync_copy(out_v, out_t_hbm.at[:, pl.ds(base, B_PER_W)])

    return k


_sc_kernel = _make_kernel()


def kernel(inputs, emb_inst_table, emb_cont_table):
    out_t = _sc_kernel(emb_inst_table.T, emb_cont_table.T,
                       inputs.T.astype(jnp.int32))
    return out_t.T
